# baseline (speedup 1.0000x reference)
import jax
import jax.numpy as jnp
from jax import lax
from jax.experimental import pallas as pl
from jax.experimental.pallas import tpu as pltpu
from jax.experimental.pallas import tpu_sc as plsc

T = 4
N_SRC = 100000
N_DST = 12288
E = 1600000
HID = 128

NC = 2
NS = 16
TPC = T // NC

NSRC_P = 100096
SRC_SL = NSRC_P // NS
DSTP = N_DST + 128
DST_SL = DSTP // NS
OUT_SL = N_DST // NS

EPS = E // NS
SB = 2048
NB = EPS // SB
TAIL = EPS - NB * SB
NSLOT = 4
NR = NB // NSLOT


def _rsqrt16(d):
    bi = lax.bitcast_convert_type(d, jnp.int32)
    bi = jnp.int32(0x5F3759DF) - lax.shift_right_arithmetic(bi, 1)
    y = lax.bitcast_convert_type(bi, jnp.float32)
    for _ in range(3):
        y = y * (1.5 - 0.5 * d * y * y)
    return y


def _sc_body(x_hbm, es_hbm, ed_hbm, out_hbm,
             deg_sh, h_sh, agg_sh, din_sh, *scr):
    sidx = list(scr[0:NSLOT])
    didx = list(scr[NSLOT:2 * NSLOT])
    vals = list(scr[2 * NSLOT:3 * NSLOT])
    sidxT, didxT, valsT = scr[3 * NSLOT:3 * NSLOT + 3]
    ones_b, zbuf, xbuf, dbuf, hbuf = scr[3 * NSLOT + 3:3 * NSLOT + 8]
    base = 3 * NSLOT + 8
    sem_st = list(scr[base:base + NSLOT])
    sem_g = list(scr[base + NSLOT:base + 2 * NSLOT])
    sem_sc = list(scr[base + 2 * NSLOT:base + 3 * NSLOT])
    c = lax.axis_index("c")
    s = lax.axis_index("s")

    def _fill_z(i, _):
        zbuf[pl.ds(i * 16, 16)] = jnp.zeros((16,), jnp.float32)
        return 0
    lax.fori_loop(0, SRC_SL // 16, _fill_z, 0)

    def _fill_o(i, _):
        ones_b[pl.ds(i * 16, 16)] = jnp.ones((16,), jnp.float32)
        return 0
    lax.fori_loop(0, SB // 16, _fill_o, 0)

    def per_t(i, _):
        t = c * TPC + i
        e_base = t * E + s * EPS
        e_tail = pl.multiple_of(e_base + NB * SB, 8)
        plsc.subcore_barrier()
        pltpu.sync_copy(zbuf, deg_sh.at[pl.ds(s * SRC_SL, SRC_SL)])
        pltpu.sync_copy(zbuf.at[pl.ds(0, DST_SL)],
                        agg_sh.at[pl.ds(s * DST_SL, DST_SL)])
        pltpu.sync_copy(zbuf.at[pl.ds(0, DST_SL)],
                        din_sh.at[pl.ds(s * DST_SL, DST_SL)])
        plsc.subcore_barrier()

        def _st_s(r, blk):
            e0 = pl.multiple_of(e_base + blk * SB, 8)
            pltpu.async_copy(es_hbm.at[pl.ds(e0, SB)], sidx[r], sem_st[r])

        def _st_s_wait(r):
            pltpu.make_async_copy(es_hbm.at[pl.ds(0, SB)], sidx[r],
                                  sem_st[r]).wait()

        def _sc_wait(r):
            pltpu.make_async_copy(x_hbm.at[pl.ds(0, SB)], vals[r],
                                  sem_sc[r]).wait()

        for r in range(2):
            _st_s(r, r)

        def p1(k, _):
            for r in range(NSLOT):
                blk = k * NSLOT + r
                _st_s_wait(r)
                pltpu.async_copy(ones_b, deg_sh.at[sidx[r]], sem_sc[r],
                                 add=True)
                rw = (r - 2) % NSLOT
                if r >= 2:
                    _sc_wait(rw)
                else:
                    @pl.when(k > 0)
                    def _():
                        _sc_wait(rw)
                rf = (r + 2) % NSLOT
                if r < 2:
                    _st_s(rf, blk + 2)
                else:
                    @pl.when(k < NR - 1)
                    def _():
                        _st_s(rf, blk + 2)
            return 0
        lax.fori_loop(0, NR, p1, 0)
        _sc_wait((NB - 2) % NSLOT)
        _sc_wait((NB - 1) % NSLOT)
        pltpu.sync_copy(es_hbm.at[pl.ds(e_tail, TAIL)], sidxT)
        pltpu.sync_copy(ones_b.at[pl.ds(0, TAIL)], deg_sh.at[sidxT],
                        add=True)
        plsc.subcore_barrier()

        x_off = pl.multiple_of(t * NSRC_P + s * SRC_SL, 16)
        pltpu.sync_copy(deg_sh.at[pl.ds(s * SRC_SL, SRC_SL)], dbuf)
        pltpu.sync_copy(x_hbm.at[pl.ds(x_off, SRC_SL)], xbuf)

        def p2(k, _):
            d = jnp.maximum(dbuf[pl.ds(k * 16, 16)], 1.0)
            y = _rsqrt16(d)
            xv = xbuf[pl.ds(k * 16, 16)]
            xv = jnp.where(xv == xv, xv, 0.0)
            hbuf[pl.ds(k * 16, 16)] = xv * y
            return 0
        lax.fori_loop(0, SRC_SL // 16, p2, 0)
        pltpu.sync_copy(hbuf, h_sh.at[pl.ds(s * SRC_SL, SRC_SL)])
        plsc.subcore_barrier()

        def _st_sd(r, blk):
            e0 = pl.multiple_of(e_base + blk * SB, 8)
            pltpu.async_copy(es_hbm.at[pl.ds(e0, SB)], sidx[r], sem_st[r])
            pltpu.async_copy(ed_hbm.at[pl.ds(e0, SB)], didx[r], sem_st[r])

        def _st_sd_wait(r):
            pltpu.make_async_copy(es_hbm.at[pl.ds(0, SB)], sidx[r],
                                  sem_st[r]).wait()
            pltpu.make_async_copy(es_hbm.at[pl.ds(0, SB)], didx[r],
                                  sem_st[r]).wait()

        def _g_fire(r):
            pltpu.async_copy(h_sh.at[sidx[r]], vals[r], sem_g[r])

        def _g_wait(r):
            pltpu.make_async_copy(x_hbm.at[pl.ds(0, SB)], vals[r],
                                  sem_g[r]).wait()

        def _sc2_wait(r):
            _sc_wait(r)
            _sc_wait(r)

        for r in range(2):
            _st_sd(r, r)
        _st_sd_wait(0)
        _g_fire(0)

        def p3(k, _):
            for r in range(NSLOT):
                blk = k * NSLOT + r
                _g_wait(r)
                pltpu.async_copy(vals[r], agg_sh.at[didx[r]], sem_sc[r],
                                 add=True)
                pltpu.async_copy(ones_b, din_sh.at[didx[r]], sem_sc[r],
                                 add=True)
                rw = (r - 2) % NSLOT
                if r >= 2:
                    _sc2_wait(rw)
                else:
                    @pl.when(k > 0)
                    def _():
                        _sc2_wait(rw)
                rg = (r + 1) % NSLOT
                if r < NSLOT - 1:
                    _st_sd_wait(rg)
                    _g_fire(rg)
                else:
                    @pl.when(k < NR - 1)
                    def _():
                        _st_sd_wait(rg)
                        _g_fire(rg)
                rf = (r + 2) % NSLOT
                if r < 2:
                    _st_sd(rf, blk + 2)
                else:
                    @pl.when(k < NR - 1)
                    def _():
                        _st_sd(rf, blk + 2)
            return 0
        lax.fori_loop(0, NR, p3, 0)
        _sc2_wait((NB - 2) % NSLOT)
        _sc2_wait((NB - 1) % NSLOT)
        pltpu.sync_copy(es_hbm.at[pl.ds(e_tail, TAIL)], sidxT)
        pltpu.sync_copy(ed_hbm.at[pl.ds(e_tail, TAIL)], didxT)
        pltpu.sync_copy(h_sh.at[sidxT], valsT)
        pltpu.sync_copy(valsT, agg_sh.at[didxT], add=True)
        pltpu.sync_copy(ones_b.at[pl.ds(0, TAIL)], din_sh.at[didxT],
                        add=True)
        plsc.subcore_barrier()

        o_agg = pl.multiple_of(t * N_DST + s * OUT_SL, 16)
        o_din = pl.multiple_of((T + t) * N_DST + s * OUT_SL, 16)
        pltpu.sync_copy(agg_sh.at[pl.ds(s * OUT_SL, OUT_SL)],
                        hbuf.at[pl.ds(0, OUT_SL)])
        pltpu.sync_copy(hbuf.at[pl.ds(0, OUT_SL)],
                        out_hbm.at[pl.ds(o_agg, OUT_SL)])
        pltpu.sync_copy(din_sh.at[pl.ds(s * OUT_SL, OUT_SL)],
                        hbuf.at[pl.ds(0, OUT_SL)])
        pltpu.sync_copy(hbuf.at[pl.ds(0, OUT_SL)],
                        out_hbm.at[pl.ds(o_din, OUT_SL)])
        return 0

    lax.fori_loop(0, TPC, per_t, 0)


def _sc_call(x2, es, ed):
    mesh = plsc.VectorSubcoreMesh(
        core_axis_name="c", subcore_axis_name="s",
        num_cores=NC, num_subcores=NS)
    f = pl.kernel(
        _sc_body,
        out_type=jax.ShapeDtypeStruct((2 * T * N_DST,), jnp.float32),
        mesh=mesh,
        scratch_types=[
            pltpu.VMEM_SHARED((NSRC_P,), jnp.float32),
            pltpu.VMEM_SHARED((NSRC_P,), jnp.float32),
            pltpu.VMEM_SHARED((DSTP,), jnp.float32),
            pltpu.VMEM_SHARED((DSTP,), jnp.float32),
            *[pltpu.VMEM((SB,), jnp.int32) for _ in range(NSLOT)],
            *[pltpu.VMEM((SB,), jnp.int32) for _ in range(NSLOT)],
            *[pltpu.VMEM((SB,), jnp.float32) for _ in range(NSLOT)],
            pltpu.VMEM((TAIL,), jnp.int32),
            pltpu.VMEM((TAIL,), jnp.int32),
            pltpu.VMEM((TAIL,), jnp.float32),
            pltpu.VMEM((SB,), jnp.float32),
            pltpu.VMEM((SRC_SL,), jnp.float32),
            pltpu.VMEM((SRC_SL,), jnp.float32),
            pltpu.VMEM((SRC_SL,), jnp.float32),
            pltpu.VMEM((SRC_SL,), jnp.float32),
            *[pltpu.SemaphoreType.DMA for _ in range(NSLOT)],
            *[pltpu.SemaphoreType.DMA for _ in range(NSLOT)],
            *[pltpu.SemaphoreType.DMA for _ in range(NSLOT)],
        ],
    )
    return f(x2, es, ed)


def _tc_body(part_ref, w_ref, b_ref, out_ref):
    p = part_ref[...]
    agg = p[0:T, :]
    din = jnp.maximum(p[T:2 * T, :], 1.0)
    sc = agg * lax.rsqrt(din)
    st = sc.T
    y = st[:, :, None] * w_ref[...][None, :, :] + b_ref[...][None, :, :]
    y = jnp.where(y > 0, y, 0.01 * y)
    out_ref[...] = y[:, None, :, :]


def _tc_call(part, W2, b):
    BD = 1024
    grid = (N_DST // BD,)
    return pl.pallas_call(
        _tc_body,
        grid=grid,
        in_specs=[
            pl.BlockSpec((2 * T, BD), lambda i: (0, i)),
            pl.BlockSpec((T, HID), lambda i: (0, 0)),
            pl.BlockSpec((T, HID), lambda i: (0, 0)),
        ],
        out_specs=pl.BlockSpec((BD, 1, T, HID), lambda i: (i, 0, 0, 0)),
        out_shape=jax.ShapeDtypeStruct((N_DST, 1, T, HID), jnp.float32),
    )(part, W2, b)


def kernel(x, edge_src, edge_dst, W, b):
    x2 = jnp.pad(x[..., 0], ((0, 0), (0, NSRC_P - N_SRC))).reshape(-1)
    es = edge_src.astype(jnp.int32).reshape(-1)
    ed = edge_dst.astype(jnp.int32).reshape(-1)

    part = _sc_call(x2, es, ed).reshape(2 * T, N_DST)
    W2 = W.reshape(T, HID)
    return _tc_call(part, W2, b)


# speedup vs baseline: 2.5370x; 2.5370x over previous
import jax
import jax.numpy as jnp
from jax import lax
from jax.experimental import pallas as pl
from jax.experimental.pallas import tpu as pltpu
from jax.experimental.pallas import tpu_sc as plsc

T = 4
N_SRC = 100000
N_DST = 12288
E = 1600000
HID = 128

NC = 2
NS = 16
TPC = T // NC

NSRC_P = 100096
SRC_SL = NSRC_P // NS
DSTP = N_DST + 128
DST_SL = DSTP // NS
OUT_SL = N_DST // NS

E_P = 1638400
EPS = E_P // NS
SB = 2048
NB = EPS // SB
NSLOT = 5
NR = NB // NSLOT


def _rsqrt16(d):
    bi = lax.bitcast_convert_type(d, jnp.int32)
    bi = jnp.int32(0x5F3759DF) - lax.shift_right_arithmetic(bi, 1)
    y = lax.bitcast_convert_type(bi, jnp.float32)
    for _ in range(3):
        y = y * (1.5 - 0.5 * d * y * y)
    return y


def _sc_body(x_hbm, es_hbm, ed_hbm, out_hbm,
             deg_sh, h_sh, agg_sh, din_sh, *scr):
    sidx = list(scr[0:NSLOT])
    didx = list(scr[NSLOT:2 * NSLOT])
    vals = list(scr[2 * NSLOT:3 * NSLOT])
    ones_b, zbuf, xbuf, dbuf, hbuf = scr[3 * NSLOT:3 * NSLOT + 5]
    base = 3 * NSLOT + 5
    sem_st = list(scr[base:base + NSLOT])
    sem_g = list(scr[base + NSLOT:base + 2 * NSLOT])
    sem_sc = list(scr[base + 2 * NSLOT:base + 3 * NSLOT])
    c = lax.axis_index("c")
    s = lax.axis_index("s")

    def _fill_z(i, _):
        zbuf[pl.ds(i * 16, 16)] = jnp.zeros((16,), jnp.float32)
        return 0
    lax.fori_loop(0, SRC_SL // 16, _fill_z, 0)

    def _fill_o(i, _):
        ones_b[pl.ds(i * 16, 16)] = jnp.ones((16,), jnp.float32)
        return 0
    lax.fori_loop(0, SB // 16, _fill_o, 0)

    def per_t(i, _):
        t = c * TPC + i
        e_base = t * E_P + s * EPS
        plsc.subcore_barrier()
        pltpu.sync_copy(zbuf, deg_sh.at[pl.ds(s * SRC_SL, SRC_SL)])
        pltpu.sync_copy(zbuf.at[pl.ds(0, DST_SL)],
                        agg_sh.at[pl.ds(s * DST_SL, DST_SL)])
        pltpu.sync_copy(zbuf.at[pl.ds(0, DST_SL)],
                        din_sh.at[pl.ds(s * DST_SL, DST_SL)])
        plsc.subcore_barrier()

        def _st_s(r, blk):
            e0 = pl.multiple_of(e_base + blk * SB, 8)
            pltpu.async_copy(es_hbm.at[pl.ds(e0, SB)], sidx[r], sem_st[r])

        def _st_s_wait(r):
            pltpu.make_async_copy(es_hbm.at[pl.ds(0, SB)], sidx[r],
                                  sem_st[r]).wait()

        def _sc_wait(r):
            pltpu.make_async_copy(x_hbm.at[pl.ds(0, SB)], vals[r],
                                  sem_sc[r]).wait()

        for r in range(3):
            _st_s(r, r)

        def p1(k, _):
            for r in range(NSLOT):
                blk = k * NSLOT + r
                _st_s_wait(r)
                pltpu.async_copy(ones_b, deg_sh.at[sidx[r]], sem_sc[r],
                                 add=True)
                rw = (r - 2) % NSLOT
                if r >= 2:
                    _sc_wait(rw)
                else:
                    @pl.when(k > 0)
                    def _():
                        _sc_wait(rw)
                rf = (r + 3) % NSLOT
                if r < 2:
                    _st_s(rf, blk + 3)
                else:
                    @pl.when(k < NR - 1)
                    def _():
                        _st_s(rf, blk + 3)
            return 0
        lax.fori_loop(0, NR, p1, 0)
        _sc_wait((NB - 2) % NSLOT)
        _sc_wait((NB - 1) % NSLOT)
        plsc.subcore_barrier()

        x_off = pl.multiple_of(t * NSRC_P + s * SRC_SL, 16)
        pltpu.sync_copy(deg_sh.at[pl.ds(s * SRC_SL, SRC_SL)], dbuf)
        pltpu.sync_copy(x_hbm.at[pl.ds(x_off, SRC_SL)], xbuf)

        def p2(k, _):
            d = jnp.maximum(dbuf[pl.ds(k * 16, 16)], 1.0)
            y = _rsqrt16(d)
            xv = xbuf[pl.ds(k * 16, 16)]
            xv = jnp.where(xv == xv, xv, 0.0)
            hbuf[pl.ds(k * 16, 16)] = xv * y
            return 0
        lax.fori_loop(0, SRC_SL // 16, p2, 0)
        pltpu.sync_copy(hbuf, h_sh.at[pl.ds(s * SRC_SL, SRC_SL)])
        plsc.subcore_barrier()

        def _st_sd(r, blk):
            e0 = pl.multiple_of(e_base + blk * SB, 8)
            pltpu.async_copy(es_hbm.at[pl.ds(e0, SB)], sidx[r], sem_st[r])
            pltpu.async_copy(ed_hbm.at[pl.ds(e0, SB)], didx[r], sem_st[r])

        def _st_sd_wait(r):
            pltpu.make_async_copy(es_hbm.at[pl.ds(0, SB)], sidx[r],
                                  sem_st[r]).wait()
            pltpu.make_async_copy(es_hbm.at[pl.ds(0, SB)], didx[r],
                                  sem_st[r]).wait()

        def _g_fire(r):
            pltpu.async_copy(h_sh.at[sidx[r]], vals[r], sem_g[r])

        def _g_wait(r):
            pltpu.make_async_copy(x_hbm.at[pl.ds(0, SB)], vals[r],
                                  sem_g[r]).wait()

        def _sc2_wait(r):
            _sc_wait(r)
            _sc_wait(r)

        for r in range(3):
            _st_sd(r, r)
        _st_sd_wait(0)
        _g_fire(0)

        def p3(k, _):
            for r in range(NSLOT):
                blk = k * NSLOT + r
                _g_wait(r)
                pltpu.async_copy(vals[r], agg_sh.at[didx[r]], sem_sc[r],
                                 add=True)
                pltpu.async_copy(ones_b, din_sh.at[didx[r]], sem_sc[r],
                                 add=True)
                rw = (r - 2) % NSLOT
                if r >= 2:
                    _sc2_wait(rw)
                else:
                    @pl.when(k > 0)
                    def _():
                        _sc2_wait(rw)
                rg = (r + 1) % NSLOT
                if r < NSLOT - 1:
                    _st_sd_wait(rg)
                    _g_fire(rg)
                else:
                    @pl.when(k < NR - 1)
                    def _():
                        _st_sd_wait(rg)
                        _g_fire(rg)
                rf = (r + 3) % NSLOT
                if r < 2:
                    _st_sd(rf, blk + 3)
                else:
                    @pl.when(k < NR - 1)
                    def _():
                        _st_sd(rf, blk + 3)
            return 0
        lax.fori_loop(0, NR, p3, 0)
        _sc2_wait((NB - 2) % NSLOT)
        _sc2_wait((NB - 1) % NSLOT)
        plsc.subcore_barrier()

        o_agg = pl.multiple_of(t * N_DST + s * OUT_SL, 16)
        o_din = pl.multiple_of((T + t) * N_DST + s * OUT_SL, 16)
        pltpu.sync_copy(agg_sh.at[pl.ds(s * OUT_SL, OUT_SL)],
                        hbuf.at[pl.ds(0, OUT_SL)])
        pltpu.sync_copy(hbuf.at[pl.ds(0, OUT_SL)],
                        out_hbm.at[pl.ds(o_agg, OUT_SL)])
        pltpu.sync_copy(din_sh.at[pl.ds(s * OUT_SL, OUT_SL)],
                        hbuf.at[pl.ds(0, OUT_SL)])
        pltpu.sync_copy(hbuf.at[pl.ds(0, OUT_SL)],
                        out_hbm.at[pl.ds(o_din, OUT_SL)])
        return 0

    lax.fori_loop(0, TPC, per_t, 0)


def _sc_call(x2, es, ed):
    mesh = plsc.VectorSubcoreMesh(
        core_axis_name="c", subcore_axis_name="s",
        num_cores=NC, num_subcores=NS)
    f = pl.kernel(
        _sc_body,
        out_type=jax.ShapeDtypeStruct((2 * T * N_DST,), jnp.float32),
        mesh=mesh,
        scratch_types=[
            pltpu.VMEM_SHARED((NSRC_P,), jnp.float32),
            pltpu.VMEM_SHARED((NSRC_P,), jnp.float32),
            pltpu.VMEM_SHARED((DSTP,), jnp.float32),
            pltpu.VMEM_SHARED((DSTP,), jnp.float32),
            *[pltpu.VMEM((SB,), jnp.int32) for _ in range(NSLOT)],
            *[pltpu.VMEM((SB,), jnp.int32) for _ in range(NSLOT)],
            *[pltpu.VMEM((SB,), jnp.float32) for _ in range(NSLOT)],
            pltpu.VMEM((SB,), jnp.float32),
            pltpu.VMEM((SRC_SL,), jnp.float32),
            pltpu.VMEM((SRC_SL,), jnp.float32),
            pltpu.VMEM((SRC_SL,), jnp.float32),
            pltpu.VMEM((SRC_SL,), jnp.float32),
            *[pltpu.SemaphoreType.DMA for _ in range(NSLOT)],
            *[pltpu.SemaphoreType.DMA for _ in range(NSLOT)],
            *[pltpu.SemaphoreType.DMA for _ in range(NSLOT)],
        ],
    )
    return f(x2, es, ed)


def _tc_body(part_ref, w_ref, b_ref, out_ref):
    p = part_ref[...]
    agg = p[0:T, :]
    din = jnp.maximum(p[T:2 * T, :], 1.0)
    sc = agg * lax.rsqrt(din)
    st = sc.T
    y = st[:, :, None] * w_ref[...][None, :, :] + b_ref[...][None, :, :]
    y = jnp.where(y > 0, y, 0.01 * y)
    out_ref[...] = y[:, None, :, :]


def _tc_call(part, W2, b):
    BD = 1024
    grid = (N_DST // BD,)
    return pl.pallas_call(
        _tc_body,
        grid=grid,
        in_specs=[
            pl.BlockSpec((2 * T, BD), lambda i: (0, i)),
            pl.BlockSpec((T, HID), lambda i: (0, 0)),
            pl.BlockSpec((T, HID), lambda i: (0, 0)),
        ],
        out_specs=pl.BlockSpec((BD, 1, T, HID), lambda i: (i, 0, 0, 0)),
        out_shape=jax.ShapeDtypeStruct((N_DST, 1, T, HID), jnp.float32),
    )(part, W2, b)


def kernel(x, edge_src, edge_dst, W, b):
    x2 = jnp.pad(x[..., 0], ((0, 0), (0, NSRC_P - N_SRC))).reshape(-1)
    pad_n = E_P - E
    src_pad = (N_SRC + (jnp.arange(pad_n, dtype=jnp.int32) % (NSRC_P - N_SRC)))
    dst_pad = (N_DST + (jnp.arange(pad_n, dtype=jnp.int32) % (DSTP - N_DST)))
    es = jnp.concatenate(
        [edge_src.astype(jnp.int32),
         jnp.broadcast_to(src_pad[None], (T, pad_n))], axis=1).reshape(-1)
    ed = jnp.concatenate(
        [edge_dst.astype(jnp.int32),
         jnp.broadcast_to(dst_pad[None], (T, pad_n))], axis=1).reshape(-1)

    part = _sc_call(x2, es, ed).reshape(2 * T, N_DST)
    W2 = W.reshape(T, HID)
    return _tc_call(part, W2, b)


# speedup vs baseline: 2.6684x; 1.0518x over previous
import jax
import jax.numpy as jnp
from jax import lax
from jax.experimental import pallas as pl
from jax.experimental.pallas import tpu as pltpu
from jax.experimental.pallas import tpu_sc as plsc

T = 4
N_SRC = 100000
N_DST = 12288
E = 1600000
HID = 128

NC = 2
NS = 16
TPC = T // NC

NSRC_P = 100096
SRC_SL = NSRC_P // NS
DSTP = N_DST + 128
DST_SL = DSTP // NS
OUT_SL = N_DST // NS

E_P = 1638400
EPS = E_P // NS
SB = 4096
NB = EPS // SB
NSLOT = 5
NR = NB // NSLOT


def _rsqrt16(d):
    bi = lax.bitcast_convert_type(d, jnp.int32)
    bi = jnp.int32(0x5F3759DF) - lax.shift_right_arithmetic(bi, 1)
    y = lax.bitcast_convert_type(bi, jnp.float32)
    for _ in range(3):
        y = y * (1.5 - 0.5 * d * y * y)
    return y


def _sc_body(x_hbm, es_hbm, ed_hbm, out_hbm,
             deg_sh, h_sh, agg_sh, din_sh, *scr):
    sidx = list(scr[0:NSLOT])
    didx = list(scr[NSLOT:2 * NSLOT])
    vals = list(scr[2 * NSLOT:3 * NSLOT])
    ones_b, zbuf, xbuf, dbuf, hbuf = scr[3 * NSLOT:3 * NSLOT + 5]
    base = 3 * NSLOT + 5
    sem_st = list(scr[base:base + NSLOT])
    sem_g = list(scr[base + NSLOT:base + 2 * NSLOT])
    sem_sc = list(scr[base + 2 * NSLOT:base + 3 * NSLOT])
    c = lax.axis_index("c")
    s = lax.axis_index("s")

    def _fill_z(i, _):
        zbuf[pl.ds(i * 16, 16)] = jnp.zeros((16,), jnp.float32)
        return 0
    lax.fori_loop(0, SRC_SL // 16, _fill_z, 0)

    def _fill_o(i, _):
        ones_b[pl.ds(i * 16, 16)] = jnp.ones((16,), jnp.float32)
        return 0
    lax.fori_loop(0, SB // 16, _fill_o, 0)


    def per_t(i, _):
        t = c * TPC + i
        e_base = t * E_P + s * EPS
        plsc.subcore_barrier()
        pltpu.sync_copy(zbuf, deg_sh.at[pl.ds(s * SRC_SL, SRC_SL)])
        pltpu.sync_copy(zbuf.at[pl.ds(0, DST_SL)],
                        agg_sh.at[pl.ds(s * DST_SL, DST_SL)])
        pltpu.sync_copy(zbuf.at[pl.ds(0, DST_SL)],
                        din_sh.at[pl.ds(s * DST_SL, DST_SL)])
        plsc.subcore_barrier()

        def _st_s(r, blk):
            e0 = pl.multiple_of(e_base + blk * SB, 8)
            pltpu.async_copy(es_hbm.at[pl.ds(e0, SB)], sidx[r], sem_st[r])

        def _st_s_wait(r):
            pltpu.make_async_copy(es_hbm.at[pl.ds(0, SB)], sidx[r],
                                  sem_st[r]).wait()

        def _sc_wait(r):
            pltpu.make_async_copy(x_hbm.at[pl.ds(0, SB)], vals[r],
                                  sem_sc[r]).wait()

        for r in range(3):
            _st_s(r, r)

        def p1(k, _):
            for r in range(NSLOT):
                blk = k * NSLOT + r
                _st_s_wait(r)
                pltpu.async_copy(ones_b, deg_sh.at[sidx[r]], sem_sc[r],
                                 add=True)
                rw = (r - 2) % NSLOT
                if r >= 2:
                    _sc_wait(rw)
                else:
                    @pl.when(k > 0)
                    def _():
                        _sc_wait(rw)
                rf = (r + 3) % NSLOT
                if r < 2:
                    _st_s(rf, blk + 3)
                else:
                    @pl.when(k < NR - 1)
                    def _():
                        _st_s(rf, blk + 3)
            return 0
        lax.fori_loop(0, NR, p1, 0)
        _sc_wait((NB - 2) % NSLOT)
        _sc_wait((NB - 1) % NSLOT)
        plsc.subcore_barrier()

        x_off = pl.multiple_of(t * NSRC_P + s * SRC_SL, 16)
        pltpu.sync_copy(deg_sh.at[pl.ds(s * SRC_SL, SRC_SL)], dbuf)
        pltpu.sync_copy(x_hbm.at[pl.ds(x_off, SRC_SL)], xbuf)

        def p2(k, _):
            d = jnp.maximum(dbuf[pl.ds(k * 16, 16)], 1.0)
            y = _rsqrt16(d)
            xv = xbuf[pl.ds(k * 16, 16)]
            xv = jnp.where(xv == xv, xv, 0.0)
            hbuf[pl.ds(k * 16, 16)] = xv * y
            return 0
        lax.fori_loop(0, SRC_SL // 16, p2, 0)
        pltpu.sync_copy(hbuf, h_sh.at[pl.ds(s * SRC_SL, SRC_SL)])
        plsc.subcore_barrier()

        def _st_sd(r, blk):
            e0 = pl.multiple_of(e_base + blk * SB, 8)
            pltpu.async_copy(es_hbm.at[pl.ds(e0, SB)], sidx[r], sem_st[r])
            pltpu.async_copy(ed_hbm.at[pl.ds(e0, SB)], didx[r], sem_st[r])

        def _st_sd_wait(r):
            pltpu.make_async_copy(es_hbm.at[pl.ds(0, SB)], sidx[r],
                                  sem_st[r]).wait()
            pltpu.make_async_copy(es_hbm.at[pl.ds(0, SB)], didx[r],
                                  sem_st[r]).wait()

        def _g_fire(r):
            pltpu.async_copy(h_sh.at[sidx[r]], vals[r], sem_g[r])

        def _g_wait(r):
            pltpu.make_async_copy(x_hbm.at[pl.ds(0, SB)], vals[r],
                                  sem_g[r]).wait()

        def _sc2_wait(r):
            _sc_wait(r)
            _sc_wait(r)

        for r in range(3):
            _st_sd(r, r)
        _st_sd_wait(0)
        _g_fire(0)

        def p3(k, _):
            for r in range(NSLOT):
                blk = k * NSLOT + r
                _g_wait(r)
                pltpu.async_copy(vals[r], agg_sh.at[didx[r]], sem_sc[r],
                                 add=True)
                pltpu.async_copy(ones_b, din_sh.at[didx[r]], sem_sc[r],
                                 add=True)
                rw = (r - 2) % NSLOT
                if r >= 2:
                    _sc2_wait(rw)
                else:
                    @pl.when(k > 0)
                    def _():
                        _sc2_wait(rw)
                rg = (r + 1) % NSLOT
                if r < NSLOT - 1:
                    _st_sd_wait(rg)
                    _g_fire(rg)
                else:
                    @pl.when(k < NR - 1)
                    def _():
                        _st_sd_wait(rg)
                        _g_fire(rg)
                rf = (r + 3) % NSLOT
                if r < 2:
                    _st_sd(rf, blk + 3)
                else:
                    @pl.when(k < NR - 1)
                    def _():
                        _st_sd(rf, blk + 3)
            return 0
        lax.fori_loop(0, NR, p3, 0)
        _sc2_wait((NB - 2) % NSLOT)
        _sc2_wait((NB - 1) % NSLOT)
        plsc.subcore_barrier()

        o_agg = pl.multiple_of(t * N_DST + s * OUT_SL, 16)
        o_din = pl.multiple_of((T + t) * N_DST + s * OUT_SL, 16)
        pltpu.sync_copy(agg_sh.at[pl.ds(s * OUT_SL, OUT_SL)],
                        hbuf.at[pl.ds(0, OUT_SL)])
        pltpu.sync_copy(hbuf.at[pl.ds(0, OUT_SL)],
                        out_hbm.at[pl.ds(o_agg, OUT_SL)])
        pltpu.sync_copy(din_sh.at[pl.ds(s * OUT_SL, OUT_SL)],
                        hbuf.at[pl.ds(0, OUT_SL)])
        pltpu.sync_copy(hbuf.at[pl.ds(0, OUT_SL)],
                        out_hbm.at[pl.ds(o_din, OUT_SL)])
        return 0

    lax.fori_loop(0, TPC, per_t, 0)


def _sc_call(x2, es, ed):
    mesh = plsc.VectorSubcoreMesh(
        core_axis_name="c", subcore_axis_name="s",
        num_cores=NC, num_subcores=NS)
    f = pl.kernel(
        _sc_body,
        out_type=jax.ShapeDtypeStruct((2 * T * N_DST,), jnp.float32),
        mesh=mesh,
        scratch_types=[
            pltpu.VMEM_SHARED((NSRC_P,), jnp.float32),
            pltpu.VMEM_SHARED((NSRC_P,), jnp.float32),
            pltpu.VMEM_SHARED((DSTP,), jnp.float32),
            pltpu.VMEM_SHARED((DSTP,), jnp.float32),
            *[pltpu.VMEM((SB,), jnp.int32) for _ in range(NSLOT)],
            *[pltpu.VMEM((SB,), jnp.int32) for _ in range(NSLOT)],
            *[pltpu.VMEM((SB,), jnp.float32) for _ in range(NSLOT)],
            pltpu.VMEM((SB,), jnp.float32),
            pltpu.VMEM((SRC_SL,), jnp.float32),
            pltpu.VMEM((SRC_SL,), jnp.float32),
            pltpu.VMEM((SRC_SL,), jnp.float32),
            pltpu.VMEM((SRC_SL,), jnp.float32),
            *[pltpu.SemaphoreType.DMA for _ in range(NSLOT)],
            *[pltpu.SemaphoreType.DMA for _ in range(NSLOT)],
            *[pltpu.SemaphoreType.DMA for _ in range(NSLOT)],
        ],
    )
    return f(x2, es, ed)


def _tc_body(part_ref, w_ref, b_ref, out_ref):
    p = part_ref[...]
    agg = p[0:T, :]
    din = jnp.maximum(p[T:2 * T, :], 1.0)
    sc = agg * lax.rsqrt(din)
    st = sc.T
    y = st[:, :, None] * w_ref[...][None, :, :] + b_ref[...][None, :, :]
    y = jnp.where(y > 0, y, 0.01 * y)
    out_ref[...] = y[:, None, :, :]


def _tc_call(part, W2, b):
    BD = 1024
    grid = (N_DST // BD,)
    return pl.pallas_call(
        _tc_body,
        grid=grid,
        in_specs=[
            pl.BlockSpec((2 * T, BD), lambda i: (0, i)),
            pl.BlockSpec((T, HID), lambda i: (0, 0)),
            pl.BlockSpec((T, HID), lambda i: (0, 0)),
        ],
        out_specs=pl.BlockSpec((BD, 1, T, HID), lambda i: (i, 0, 0, 0)),
        out_shape=jax.ShapeDtypeStruct((N_DST, 1, T, HID), jnp.float32),
    )(part, W2, b)


def kernel(x, edge_src, edge_dst, W, b):
    x2 = jnp.pad(x[..., 0], ((0, 0), (0, NSRC_P - N_SRC))).reshape(-1)
    pad_n = E_P - E
    src_pad = (N_SRC + (jnp.arange(pad_n, dtype=jnp.int32) % (NSRC_P - N_SRC)))
    dst_pad = (N_DST + (jnp.arange(pad_n, dtype=jnp.int32) % (DSTP - N_DST)))
    es = jnp.concatenate(
        [edge_src.astype(jnp.int32),
         jnp.broadcast_to(src_pad[None], (T, pad_n))], axis=1).reshape(-1)
    ed = jnp.concatenate(
        [edge_dst.astype(jnp.int32),
         jnp.broadcast_to(dst_pad[None], (T, pad_n))], axis=1).reshape(-1)

    part = _sc_call(x2, es, ed).reshape(2 * T, N_DST)
    W2 = W.reshape(T, HID)
    return _tc_call(part, W2, b)


# speedup vs baseline: 3.1231x; 1.1704x over previous
import jax
import jax.numpy as jnp
from jax import lax
from jax.experimental import pallas as pl
from jax.experimental.pallas import tpu as pltpu
from jax.experimental.pallas import tpu_sc as plsc

T = 4
N_SRC = 100000
N_DST = 12288
E = 1600000
HID = 128

NC = 2
NS = 16
TPC = T // NC

NSRC_P = 100096
SRC_SL = NSRC_P // NS
DSTP = N_DST + 128
DST_SL = DSTP // NS
OUT_SL = N_DST // NS

E_P = 1638400
EPS = E_P // NS
SB = 5120
NB = EPS // SB
NSLOT = 5
NR = NB // NSLOT


def _rsqrt16(d):
    bi = lax.bitcast_convert_type(d, jnp.int32)
    bi = jnp.int32(0x5F3759DF) - lax.shift_right_arithmetic(bi, 1)
    y = lax.bitcast_convert_type(bi, jnp.float32)
    for _ in range(3):
        y = y * (1.5 - 0.5 * d * y * y)
    return y


def _sc_body(x_hbm, es_hbm, ed_hbm, out_hbm,
             deg_sh, h_sh, agg_sh, din_sh, *scr):
    sidx = list(scr[0:NSLOT])
    didx = list(scr[NSLOT:2 * NSLOT])
    vals = list(scr[2 * NSLOT:3 * NSLOT])
    ones_b, zbuf, xbuf, dbuf, hbuf = scr[3 * NSLOT:3 * NSLOT + 5]
    base = 3 * NSLOT + 5
    sem_st = list(scr[base:base + NSLOT])
    sem_g = list(scr[base + NSLOT:base + 2 * NSLOT])
    sem_sc = list(scr[base + 2 * NSLOT:base + 3 * NSLOT])
    c = lax.axis_index("c")
    s = lax.axis_index("s")

    def _fill_z(i, _):
        zbuf[pl.ds(i * 16, 16)] = jnp.zeros((16,), jnp.float32)
        return 0
    lax.fori_loop(0, SRC_SL // 16, _fill_z, 0)

    def _fill_o(i, _):
        ones_b[pl.ds(i * 16, 16)] = jnp.ones((16,), jnp.float32)
        return 0
    lax.fori_loop(0, SB // 16, _fill_o, 0)


    def per_t(i, _):
        t = c * TPC + i
        e_base = t * E_P + s * EPS
        plsc.subcore_barrier()
        pltpu.sync_copy(zbuf, deg_sh.at[pl.ds(s * SRC_SL, SRC_SL)])
        pltpu.sync_copy(zbuf.at[pl.ds(0, DST_SL)],
                        agg_sh.at[pl.ds(s * DST_SL, DST_SL)])
        pltpu.sync_copy(zbuf.at[pl.ds(0, DST_SL)],
                        din_sh.at[pl.ds(s * DST_SL, DST_SL)])
        plsc.subcore_barrier()

        def _st_s(r, blk):
            e0 = pl.multiple_of(e_base + blk * SB, 8)
            pltpu.async_copy(es_hbm.at[pl.ds(e0, SB)], sidx[r], sem_st[r])

        def _st_s_wait(r):
            pltpu.make_async_copy(es_hbm.at[pl.ds(0, SB)], sidx[r],
                                  sem_st[r]).wait()

        def _sc_wait(r):
            pltpu.make_async_copy(x_hbm.at[pl.ds(0, SB)], vals[r],
                                  sem_sc[r]).wait()

        for r in range(3):
            _st_s(r, r)

        def p1(k, _):
            for r in range(NSLOT):
                blk = k * NSLOT + r
                _st_s_wait(r)
                pltpu.async_copy(ones_b, deg_sh.at[sidx[r]], sem_sc[r],
                                 add=True)
                rw = (r - 2) % NSLOT
                if r >= 2:
                    _sc_wait(rw)
                else:
                    @pl.when(k > 0)
                    def _():
                        _sc_wait(rw)
                rf = (r + 3) % NSLOT
                if r < 2:
                    _st_s(rf, blk + 3)
                else:
                    @pl.when(k < NR - 1)
                    def _():
                        _st_s(rf, blk + 3)
            return 0
        lax.fori_loop(0, NR, p1, 0)
        _sc_wait((NB - 2) % NSLOT)
        _sc_wait((NB - 1) % NSLOT)
        plsc.subcore_barrier()

        x_off = pl.multiple_of(t * NSRC_P + s * SRC_SL, 16)
        pltpu.sync_copy(deg_sh.at[pl.ds(s * SRC_SL, SRC_SL)], dbuf)
        pltpu.sync_copy(x_hbm.at[pl.ds(x_off, SRC_SL)], xbuf)

        def p2(k, _):
            d = jnp.maximum(dbuf[pl.ds(k * 16, 16)], 1.0)
            y = _rsqrt16(d)
            xv = xbuf[pl.ds(k * 16, 16)]
            xv = jnp.where(xv == xv, xv, 0.0)
            hbuf[pl.ds(k * 16, 16)] = xv * y
            return 0
        lax.fori_loop(0, SRC_SL // 16, p2, 0)
        pltpu.sync_copy(hbuf, h_sh.at[pl.ds(s * SRC_SL, SRC_SL)])
        plsc.subcore_barrier()

        def _st_sd(r, blk):
            e0 = pl.multiple_of(e_base + blk * SB, 8)
            pltpu.async_copy(es_hbm.at[pl.ds(e0, SB)], sidx[r], sem_st[r])
            pltpu.async_copy(ed_hbm.at[pl.ds(e0, SB)], didx[r], sem_st[r])

        def _st_sd_wait(r):
            pltpu.make_async_copy(es_hbm.at[pl.ds(0, SB)], sidx[r],
                                  sem_st[r]).wait()
            pltpu.make_async_copy(es_hbm.at[pl.ds(0, SB)], didx[r],
                                  sem_st[r]).wait()

        def _g_fire(r):
            pltpu.async_copy(h_sh.at[sidx[r]], vals[r], sem_g[r])

        def _g_wait(r):
            pltpu.make_async_copy(x_hbm.at[pl.ds(0, SB)], vals[r],
                                  sem_g[r]).wait()

        def _sc2_wait(r):
            _sc_wait(r)
            _sc_wait(r)

        for r in range(3):
            _st_sd(r, r)
        for r in range(2):
            _st_sd_wait(r)
            _g_fire(r)

        def p3(k, _):
            for r in range(NSLOT):
                blk = k * NSLOT + r
                _g_wait(r)
                pltpu.async_copy(vals[r], agg_sh.at[didx[r]], sem_sc[r],
                                 add=True)
                pltpu.async_copy(ones_b, din_sh.at[didx[r]], sem_sc[r],
                                 add=True)
                rw = (r - 2) % NSLOT
                if r >= 2:
                    _sc2_wait(rw)
                else:
                    @pl.when(k > 0)
                    def _():
                        _sc2_wait(rw)
                rg = (r + 2) % NSLOT
                if r < NSLOT - 2:
                    _st_sd_wait(rg)
                    _g_fire(rg)
                else:
                    @pl.when(k < NR - 1)
                    def _():
                        _st_sd_wait(rg)
                        _g_fire(rg)
                rf = (r + 3) % NSLOT
                if r < 2:
                    _st_sd(rf, blk + 3)
                else:
                    @pl.when(k < NR - 1)
                    def _():
                        _st_sd(rf, blk + 3)
            return 0
        lax.fori_loop(0, NR, p3, 0)
        _sc2_wait((NB - 2) % NSLOT)
        _sc2_wait((NB - 1) % NSLOT)
        plsc.subcore_barrier()

        o_agg = pl.multiple_of(t * N_DST + s * OUT_SL, 16)
        o_din = pl.multiple_of((T + t) * N_DST + s * OUT_SL, 16)
        pltpu.sync_copy(agg_sh.at[pl.ds(s * OUT_SL, OUT_SL)],
                        hbuf.at[pl.ds(0, OUT_SL)])
        pltpu.sync_copy(hbuf.at[pl.ds(0, OUT_SL)],
                        out_hbm.at[pl.ds(o_agg, OUT_SL)])
        pltpu.sync_copy(din_sh.at[pl.ds(s * OUT_SL, OUT_SL)],
                        hbuf.at[pl.ds(0, OUT_SL)])
        pltpu.sync_copy(hbuf.at[pl.ds(0, OUT_SL)],
                        out_hbm.at[pl.ds(o_din, OUT_SL)])
        return 0

    lax.fori_loop(0, TPC, per_t, 0)


def _sc_call(x2, es, ed):
    mesh = plsc.VectorSubcoreMesh(
        core_axis_name="c", subcore_axis_name="s",
        num_cores=NC, num_subcores=NS)
    f = pl.kernel(
        _sc_body,
        out_type=jax.ShapeDtypeStruct((2 * T * N_DST,), jnp.float32),
        mesh=mesh,
        scratch_types=[
            pltpu.VMEM_SHARED((NSRC_P,), jnp.float32),
            pltpu.VMEM_SHARED((NSRC_P,), jnp.float32),
            pltpu.VMEM_SHARED((DSTP,), jnp.float32),
            pltpu.VMEM_SHARED((DSTP,), jnp.float32),
            *[pltpu.VMEM((SB,), jnp.int32) for _ in range(NSLOT)],
            *[pltpu.VMEM((SB,), jnp.int32) for _ in range(NSLOT)],
            *[pltpu.VMEM((SB,), jnp.float32) for _ in range(NSLOT)],
            pltpu.VMEM((SB,), jnp.float32),
            pltpu.VMEM((SRC_SL,), jnp.float32),
            pltpu.VMEM((SRC_SL,), jnp.float32),
            pltpu.VMEM((SRC_SL,), jnp.float32),
            pltpu.VMEM((SRC_SL,), jnp.float32),
            *[pltpu.SemaphoreType.DMA for _ in range(NSLOT)],
            *[pltpu.SemaphoreType.DMA for _ in range(NSLOT)],
            *[pltpu.SemaphoreType.DMA for _ in range(NSLOT)],
        ],
    )
    return f(x2, es, ed)


def _tc_body(part_ref, w_ref, b_ref, out_ref):
    p = part_ref[...]
    agg = p[0:T, :]
    din = jnp.maximum(p[T:2 * T, :], 1.0)
    sc = agg * lax.rsqrt(din)
    st = sc.T
    y = st[:, :, None] * w_ref[...][None, :, :] + b_ref[...][None, :, :]
    y = jnp.where(y > 0, y, 0.01 * y)
    out_ref[...] = y[:, None, :, :]


def _tc_call(part, W2, b):
    BD = 1024
    grid = (N_DST // BD,)
    return pl.pallas_call(
        _tc_body,
        grid=grid,
        in_specs=[
            pl.BlockSpec((2 * T, BD), lambda i: (0, i)),
            pl.BlockSpec((T, HID), lambda i: (0, 0)),
            pl.BlockSpec((T, HID), lambda i: (0, 0)),
        ],
        out_specs=pl.BlockSpec((BD, 1, T, HID), lambda i: (i, 0, 0, 0)),
        out_shape=jax.ShapeDtypeStruct((N_DST, 1, T, HID), jnp.float32),
    )(part, W2, b)


def kernel(x, edge_src, edge_dst, W, b):
    x2 = jnp.pad(x[..., 0], ((0, 0), (0, NSRC_P - N_SRC))).reshape(-1)
    pad_n = E_P - E
    src_pad = (N_SRC + (jnp.arange(pad_n, dtype=jnp.int32) % (NSRC_P - N_SRC)))
    dst_pad = (N_DST + (jnp.arange(pad_n, dtype=jnp.int32) % (DSTP - N_DST)))
    es = jnp.concatenate(
        [edge_src.astype(jnp.int32),
         jnp.broadcast_to(src_pad[None], (T, pad_n))], axis=1).reshape(-1)
    ed = jnp.concatenate(
        [edge_dst.astype(jnp.int32),
         jnp.broadcast_to(dst_pad[None], (T, pad_n))], axis=1).reshape(-1)

    part = _sc_call(x2, es, ed).reshape(2 * T, N_DST)
    W2 = W.reshape(T, HID)
    return _tc_call(part, W2, b)
